# Initial kernel scaffold; baseline (speedup 1.0000x reference)
import jax
import jax.numpy as jnp
from jax.experimental import pallas as pl


def kernel(x, codebooks):
    raise NotImplementedError("write your pallas kernel here")


# speedup vs baseline: 1.0273x; 1.0273x over previous
import jax
import jax.numpy as jnp
from jax.experimental import pallas as pl
from jax.experimental.pallas import tpu as pltpu

NQ = 4
KC = 1024
D = 384
TM = 512
NTOK = 16 * 576
NT = NTOK // TM
BETA = 0.25


def _rvq_step(x_ref, cb_ref, nrm_ref, out_ref, idx_ref, ss_ref, r_ref):
    s = pl.program_id(1)
    E = cb_ref[0]
    n = nrm_ref[0]

    @pl.when(s == 0)
    def _():
        r_ref[...] = x_ref[...]

    r = r_ref[...]
    a = jnp.sum(r * r, axis=1, keepdims=True)
    dot = jax.lax.dot_general(
        r, E, (((1,), (1,)), ((), ())),
        precision=jax.lax.Precision.DEFAULT,
        preferred_element_type=jnp.float32,
    )
    dist = (a - 2.0 * dot) + n
    m = jnp.min(dist, axis=1, keepdims=True)
    iota = jax.lax.broadcasted_iota(jnp.int32, dist.shape, 1)
    idxm = jnp.min(jnp.where(dist == m, iota, KC), axis=1, keepdims=True)
    oh = (iota == idxm).astype(jnp.float32)
    quant = jax.lax.dot_general(
        oh, E, (((1,), (0,)), ((), ())),
        precision=jax.lax.Precision.HIGHEST,
        preferred_element_type=jnp.float32,
    )
    r_new = r - quant
    r_ref[...] = r_new
    idx_ref[0, :, :] = idxm
    part = jnp.full((1, 8, 128), jnp.sum(r_new * r_new), jnp.float32)

    @pl.when(pl.program_id(0) == 0)
    def _():
        ss_ref[...] = part

    @pl.when(pl.program_id(0) != 0)
    def _():
        ss_ref[...] += part

    @pl.when(s == NQ - 1)
    def _():
        out_ref[...] = x_ref[...] - r_new


def kernel(x, codebooks):
    xf = x.reshape(NTOK, D)
    nrm = jnp.sum(codebooks ** 2, axis=2)[:, None, :]
    out, idxs, ss = pl.pallas_call(
        _rvq_step,
        grid=(NT, NQ),
        in_specs=[
            pl.BlockSpec((TM, D), lambda t, s: (t, 0)),
            pl.BlockSpec((1, KC, D), lambda t, s: (s, 0, 0)),
            pl.BlockSpec((1, 1, KC), lambda t, s: (s, 0, 0)),
        ],
        out_specs=[
            pl.BlockSpec((TM, D), lambda t, s: (t, 0)),
            pl.BlockSpec((1, TM, 1), lambda t, s: (s * NT + t, 0, 0)),
            pl.BlockSpec((1, 8, 128), lambda t, s: (s, 0, 0)),
        ],
        out_shape=[
            jax.ShapeDtypeStruct((NTOK, D), jnp.float32),
            jax.ShapeDtypeStruct((NQ * NT, TM, 1), jnp.int32),
            jax.ShapeDtypeStruct((NQ, 8, 128), jnp.float32),
        ],
        scratch_shapes=[pltpu.VMEM((TM, D), jnp.float32)],
    )(xf, codebooks, nrm)
    quantized_out = out.reshape(x.shape)
    all_indices = idxs.reshape(NQ, NTOK).T.reshape(x.shape[0], x.shape[1], NQ)
    vql = (ss[:, 0, 0] / jnp.float32(NTOK * D)).reshape(1, NQ)
    qql = BETA * vql
    return quantized_out, all_indices, vql, qql


# speedup vs baseline: 1.4317x; 1.3937x over previous
import functools

import jax
import jax.numpy as jnp
from jax.experimental import pallas as pl
from jax.experimental.pallas import tpu as pltpu
from jax.experimental.pallas import tpu_sc as plsc

NQ = 4
KC = 1024
D = 384
TM = 512
NTOK = 16 * 576
NT = NTOK // TM
BETA = 0.25

NW = 32
BPW = NTOK // NW
GCH = 96


def _dist_step(r_ref, q_ref, cb_ref, nrm_ref, rout_ref, idx_ref, ss_ref,
               *, first):
    E = cb_ref[0]
    n = nrm_ref[0]
    if first:
        r = r_ref[...]
    else:
        r = r_ref[...] - q_ref[...]
        rout_ref[...] = r
        part = jnp.full((8, 128), jnp.sum(r * r), jnp.float32)

        @pl.when(pl.program_id(0) == 0)
        def _():
            ss_ref[...] = part

        @pl.when(pl.program_id(0) != 0)
        def _():
            ss_ref[...] += part

    a = jnp.sum(r * r, axis=1, keepdims=True)
    dot = jax.lax.dot_general(
        r, E, (((1,), (1,)), ((), ())),
        precision=jax.lax.Precision.DEFAULT,
        preferred_element_type=jnp.float32,
    )
    dist = (a - 2.0 * dot) + n
    m = jnp.min(dist, axis=1, keepdims=True)
    iota = jax.lax.broadcasted_iota(jnp.int32, dist.shape, 1)
    idx_ref[...] = jnp.min(jnp.where(dist == m, iota, KC), axis=1,
                           keepdims=True)


def _dist_call(r, q, cb_s, nrm_s, first):
    outs = [
        jax.ShapeDtypeStruct((NTOK, D), jnp.float32),
        jax.ShapeDtypeStruct((NTOK, 1), jnp.int32),
        jax.ShapeDtypeStruct((8, 128), jnp.float32),
    ]
    out_specs = [
        pl.BlockSpec((TM, D), lambda t: (t, 0)),
        pl.BlockSpec((TM, 1), lambda t: (t, 0)),
        pl.BlockSpec((8, 128), lambda t: (0, 0)),
    ]
    return pl.pallas_call(
        functools.partial(_dist_step, first=first),
        grid=(NT,),
        in_specs=[
            pl.BlockSpec((TM, D), lambda t: (t, 0)),
            pl.BlockSpec((TM, D), lambda t: (t, 0)),
            pl.BlockSpec((1, KC, D), lambda t: (0, 0, 0)),
            pl.BlockSpec((1, 1, KC), lambda t: (0, 0, 0)),
        ],
        out_specs=out_specs,
        out_shape=outs,
    )(r, q, cb_s, nrm_s)


def _final_step(x_ref, r_ref, q_ref, out_ref, ss_ref):
    r_new = r_ref[...] - q_ref[...]
    out_ref[...] = x_ref[...] - r_new
    part = jnp.full((8, 128), jnp.sum(r_new * r_new), jnp.float32)

    @pl.when(pl.program_id(0) == 0)
    def _():
        ss_ref[...] = part

    @pl.when(pl.program_id(0) != 0)
    def _():
        ss_ref[...] += part


def _sc_gather(table, idx):
    mesh = plsc.VectorSubcoreMesh(core_axis_name="c", subcore_axis_name="s")

    @functools.partial(
        pl.kernel, mesh=mesh,
        out_type=jax.ShapeDtypeStruct((NTOK, D), jnp.float32),
        scratch_types=[
            pltpu.VMEM((BPW,), jnp.int32),
            pltpu.VMEM((BPW, D), jnp.float32),
            pltpu.SemaphoreType.DMA,
        ],
    )
    def k(table_hbm, idx_hbm, out_hbm, idx_v, rows_v, sem):
        wid = jax.lax.axis_index("s") * 2 + jax.lax.axis_index("c")
        base = wid * BPW
        pltpu.sync_copy(idx_hbm.at[pl.ds(base, BPW)], idx_v)
        cps = [
            pltpu.async_copy(
                table_hbm.at[idx_v.at[pl.ds(j * GCH, GCH)]],
                rows_v.at[pl.ds(j * GCH, GCH)],
                sem,
            )
            for j in range(BPW // GCH)
        ]
        for cp in cps:
            cp.wait()
        pltpu.sync_copy(rows_v, out_hbm.at[pl.ds(base, BPW)])

    return k(table, idx)


def kernel(x, codebooks):
    xf = x.reshape(NTOK, D)
    nrm = jnp.sum(codebooks ** 2, axis=2)[:, None, :]

    zeros = jnp.zeros((NTOK, D), jnp.float32)
    r = xf
    qprev = zeros
    idxs = []
    sss = []
    for s in range(NQ):
        r, idx_s, ss_prev = _dist_call(
            r, qprev, codebooks[s:s + 1], nrm[s:s + 1], first=(s == 0))
        if s == 0:
            r = xf
        else:
            sss.append(ss_prev)
        idxs.append(idx_s)
        qprev = _sc_gather(codebooks[s], idx_s.reshape(NTOK))

    out, ss_last = pl.pallas_call(
        _final_step,
        grid=(NT,),
        in_specs=[
            pl.BlockSpec((TM, D), lambda t: (t, 0)),
            pl.BlockSpec((TM, D), lambda t: (t, 0)),
            pl.BlockSpec((TM, D), lambda t: (t, 0)),
        ],
        out_specs=[
            pl.BlockSpec((TM, D), lambda t: (t, 0)),
            pl.BlockSpec((8, 128), lambda t: (0, 0)),
        ],
        out_shape=[
            jax.ShapeDtypeStruct((NTOK, D), jnp.float32),
            jax.ShapeDtypeStruct((8, 128), jnp.float32),
        ],
    )(xf, r, qprev)
    sss.append(ss_last)

    quantized_out = out.reshape(x.shape)
    all_indices = jnp.concatenate(idxs, axis=1).reshape(
        x.shape[0], x.shape[1], NQ)
    vql = (jnp.stack([s[0, 0] for s in sss]) / jnp.float32(NTOK * D)
           ).reshape(1, NQ)
    qql = BETA * vql
    return quantized_out, all_indices, vql, qql
